# Initial kernel scaffold; baseline (speedup 1.0000x reference)
import jax
import jax.numpy as jnp
from jax.experimental import pallas as pl


def kernel(x, coords, neighbor_indices):
    raise NotImplementedError("write your pallas kernel here")


# speedup vs baseline: 11.6398x; 11.6398x over previous
import functools

import jax
import jax.numpy as jnp
from jax import lax
from jax.experimental import pallas as pl
from jax.experimental.pallas import tpu as pltpu
from jax.experimental.pallas import tpu_sc as plsc

N = 10000
F = 128
C = 4
K = 32
CC = C * C

NUM_WORKERS = 32
CHUNK = 320
STRIDE = 313
LANES = 16

BN = 400


def _full(val):
    return jnp.full((LANES,), val, dtype=jnp.int32)


def _sc_cov_kernel(coords_hbm, nbr_hbm, out_hbm, coords_v, nbr_v, out_v):
    cid = lax.axis_index("c")
    sid = lax.axis_index("s")
    wid = sid * 2 + cid
    base = jnp.minimum(wid * STRIDE, N - CHUNK)

    pltpu.sync_copy(coords_hbm, coords_v)
    pltpu.sync_copy(nbr_hbm.at[pl.ds(base * K, CHUNK * K)], nbr_v)

    lane = lax.iota(jnp.int32, LANES)
    inv = jnp.float32(1.0 / (K - 1))

    def group_body(g, _):
        node_loc = g * LANES + lane
        own4 = (base + node_loc) * C
        m = [plsc.load_gather(coords_v, [own4 + c]) for c in range(C)]
        nbr_base = node_loc * K

        def k_body(kk, acc):
            idx4 = plsc.load_gather(nbr_v, [nbr_base + kk]) * C
            d = [plsc.load_gather(coords_v, [idx4 + c]) - m[c] for c in range(C)]
            new = []
            t = 0
            for a in range(C):
                for b in range(a, C):
                    new.append(acc[t] + d[a] * d[b])
                    t += 1
            return new

        acc = lax.fori_loop(0, K, k_body, [jnp.zeros((LANES,), jnp.float32)] * 10)

        out_base = node_loc * CC
        t = 0
        for a in range(C):
            for b in range(a, C):
                v = acc[t] * inv
                t += 1
                plsc.store_scatter(out_v, [out_base + (a * C + b)], v)
                if a != b:
                    plsc.store_scatter(out_v, [out_base + (b * C + a)], v)
        return 0

    lax.fori_loop(0, CHUNK // LANES, group_body, 0)
    pltpu.sync_copy(out_v, out_hbm.at[pl.ds(base * CC, CHUNK * CC)])


_sc_cov = functools.partial(
    pl.kernel,
    out_type=jax.ShapeDtypeStruct((N * CC,), jnp.float32),
    mesh=plsc.VectorSubcoreMesh(
        core_axis_name="c", subcore_axis_name="s", num_cores=2, num_subcores=16
    ),
    compiler_params=pltpu.CompilerParams(needs_layout_passes=False),
    scratch_types=[
        pltpu.VMEM((N * C,), jnp.float32),
        pltpu.VMEM((CHUNK * K,), jnp.int32),
        pltpu.VMEM((CHUNK * CC,), jnp.float32),
    ],
)(_sc_cov_kernel)


def _tc_weight_kernel(x_ref, cov_ref, rep_ref, out_ref):
    x = x_ref[...]
    cov = cov_ref[...]
    xr = jax.lax.dot(x, rep_ref[...], precision=jax.lax.Precision.DEFAULT)
    covt = jnp.tile(cov, (1, F))
    out_ref[...] = xr * covt


def kernel(x, coords, neighbor_indices):
    cov = _sc_cov(coords.reshape(-1), neighbor_indices.reshape(-1)).reshape(N, CC)

    rep = jnp.reshape(
        jnp.eye(F, dtype=jnp.float32)[:, :, None] * jnp.ones((1, 1, CC), jnp.float32),
        (F, F * CC),
    )
    return pl.pallas_call(
        _tc_weight_kernel,
        grid=(N // BN,),
        in_specs=[
            pl.BlockSpec((BN, F), lambda i: (i, 0)),
            pl.BlockSpec((BN, CC), lambda i: (i, 0)),
            pl.BlockSpec((F, F * CC), lambda i: (0, 0)),
        ],
        out_specs=pl.BlockSpec((BN, F * CC), lambda i: (i, 0)),
        out_shape=jax.ShapeDtypeStruct((N, F * CC), jnp.float32),
    )(x, cov, rep)


# speedup vs baseline: 12.5635x; 1.0794x over previous
import functools

import jax
import jax.numpy as jnp
from jax import lax
from jax.experimental import pallas as pl
from jax.experimental.pallas import tpu as pltpu
from jax.experimental.pallas import tpu_sc as plsc

N = 10000
F = 128
C = 4
K = 32
CC = C * C

NUM_WORKERS = 32
CHUNK = 320
STRIDE = 313
LANES = 16

BN = 1000


def _full(val):
    return jnp.full((LANES,), val, dtype=jnp.int32)


def _sc_cov_kernel(coords_hbm, nbr_hbm, out_hbm, coords_v, nbr_v, out_v):
    cid = lax.axis_index("c")
    sid = lax.axis_index("s")
    wid = sid * 2 + cid
    base = jnp.minimum(wid * STRIDE, N - CHUNK)

    pltpu.sync_copy(coords_hbm, coords_v)
    pltpu.sync_copy(nbr_hbm.at[pl.ds(base * K, CHUNK * K)], nbr_v)

    lane = lax.iota(jnp.int32, LANES)
    inv = jnp.float32(1.0 / (K - 1))

    def group_body(g, _):
        node_loc = g * LANES + lane
        own4 = (base + node_loc) * C
        m = [plsc.load_gather(coords_v, [own4 + c]) for c in range(C)]
        nbr_base = node_loc * K

        acc = [jnp.zeros((LANES,), jnp.float32)] * 10
        for kk in range(K):
            idx4 = plsc.load_gather(nbr_v, [nbr_base + kk]) * C
            d = [plsc.load_gather(coords_v, [idx4 + c]) - m[c] for c in range(C)]
            t = 0
            for a in range(C):
                for b in range(a, C):
                    acc[t] = acc[t] + d[a] * d[b]
                    t += 1

        out_base = node_loc * CC
        t = 0
        for a in range(C):
            for b in range(a, C):
                v = acc[t] * inv
                t += 1
                plsc.store_scatter(out_v, [out_base + (a * C + b)], v)
                if a != b:
                    plsc.store_scatter(out_v, [out_base + (b * C + a)], v)
        return 0

    lax.fori_loop(0, CHUNK // LANES, group_body, 0)
    pltpu.sync_copy(out_v, out_hbm.at[pl.ds(base * CC, CHUNK * CC)])


_sc_cov = functools.partial(
    pl.kernel,
    out_type=jax.ShapeDtypeStruct((N * CC,), jnp.float32),
    mesh=plsc.VectorSubcoreMesh(
        core_axis_name="c", subcore_axis_name="s", num_cores=2, num_subcores=16
    ),
    compiler_params=pltpu.CompilerParams(needs_layout_passes=False),
    scratch_types=[
        pltpu.VMEM((N * C,), jnp.float32),
        pltpu.VMEM((CHUNK * K,), jnp.int32),
        pltpu.VMEM((CHUNK * CC,), jnp.float32),
    ],
)(_sc_cov_kernel)


def _tc_weight_kernel(x_ref, cov_ref, rep_ref, out_ref):
    x = x_ref[...]
    cov = cov_ref[...]
    xr = jax.lax.dot(x, rep_ref[...], precision=jax.lax.Precision.DEFAULT)
    covt = jnp.tile(cov, (1, F))
    out_ref[...] = xr * covt


def kernel(x, coords, neighbor_indices):
    cov = _sc_cov(coords.reshape(-1), neighbor_indices.reshape(-1)).reshape(N, CC)

    rep = jnp.reshape(
        jnp.eye(F, dtype=jnp.float32)[:, :, None] * jnp.ones((1, 1, CC), jnp.float32),
        (F, F * CC),
    )
    return pl.pallas_call(
        _tc_weight_kernel,
        grid=(N // BN,),
        in_specs=[
            pl.BlockSpec((BN, F), lambda i: (i, 0)),
            pl.BlockSpec((BN, CC), lambda i: (i, 0)),
            pl.BlockSpec((F, F * CC), lambda i: (0, 0)),
        ],
        out_specs=pl.BlockSpec((BN, F * CC), lambda i: (i, 0)),
        out_shape=jax.ShapeDtypeStruct((N, F * CC), jnp.float32),
    )(x, cov, rep)


# speedup vs baseline: 13.1079x; 1.0433x over previous
import functools

import jax
import jax.numpy as jnp
from jax import lax
from jax.experimental import pallas as pl
from jax.experimental.pallas import tpu as pltpu
from jax.experimental.pallas import tpu_sc as plsc

N = 10000
F = 128
C = 4
CP = 5
K = 32
CC = C * C

NUM_WORKERS = 32
CHUNK = 336
STRIDE = 312
LANES = 16

BN = 1024


def _sc_cov_kernel(coords_hbm, nbr_hbm, out_hbm, coords_v, nbr_v, out_v, dsem):
    cid = lax.axis_index("c")
    sid = lax.axis_index("s")
    wid = sid * 2 + cid
    base = jnp.minimum(wid * STRIDE, N - CHUNK)

    copies = [pltpu.async_copy(coords_hbm, coords_v, dsem)]
    for kk in range(K):
        copies.append(pltpu.async_copy(
            nbr_hbm.at[pl.ds(kk * N + base, CHUNK)],
            nbr_v.at[pl.ds(kk * CHUNK, CHUNK)], dsem))
    for cp in copies:
        cp.wait()

    lane = lax.iota(jnp.int32, LANES)
    inv = jnp.float32(1.0 / (K - 1))

    def group_body(g, _):
        own5 = (base + g * LANES + lane) * CP
        m = [plsc.load_gather(coords_v, [own5 + c]) for c in range(C)]

        acc = [jnp.zeros((LANES,), jnp.float32)] * 10
        for kk in range(K):
            idx5 = nbr_v[pl.ds(kk * CHUNK + g * LANES, LANES)] * CP
            d = [plsc.load_gather(coords_v, [idx5 + c]) - m[c] for c in range(C)]
            t = 0
            for a in range(C):
                for b in range(a, C):
                    acc[t] = acc[t] + d[a] * d[b]
                    t += 1

        t = 0
        for a in range(C):
            for b in range(a, C):
                v = acc[t] * inv
                t += 1
                out_v[pl.ds((a * C + b) * CHUNK + g * LANES, LANES)] = v
                if a != b:
                    out_v[pl.ds((b * C + a) * CHUNK + g * LANES, LANES)] = v
        return 0

    lax.fori_loop(0, CHUNK // LANES, group_body, 0)
    stores = [pltpu.async_copy(
        out_v.at[pl.ds(j * CHUNK, CHUNK)],
        out_hbm.at[pl.ds(j * N + base, CHUNK)], dsem) for j in range(CC)]
    for st in stores:
        st.wait()


_sc_cov = functools.partial(
    pl.kernel,
    out_type=jax.ShapeDtypeStruct((CC * N,), jnp.float32),
    mesh=plsc.VectorSubcoreMesh(
        core_axis_name="c", subcore_axis_name="s", num_cores=2, num_subcores=16
    ),
    compiler_params=pltpu.CompilerParams(needs_layout_passes=False),
    scratch_types=[
        pltpu.VMEM((N * CP,), jnp.float32),
        pltpu.VMEM((K * CHUNK,), jnp.int32),
        pltpu.VMEM((CC * CHUNK,), jnp.float32),
        pltpu.SemaphoreType.DMA,
    ],
)(_sc_cov_kernel)


def _tc_weight_kernel(x_ref, cov_ref, rep_ref, sel_ref, out_ref):
    x = x_ref[...]
    covt = cov_ref[...]
    xr = jax.lax.dot(x, rep_ref[...], precision=jax.lax.Precision.DEFAULT)
    covq = jax.lax.dot_general(
        covt, sel_ref[...],
        dimension_numbers=(((0,), (0,)), ((), ())),
        precision=jax.lax.Precision.DEFAULT,
    )
    out_ref[...] = xr * covq


def kernel(x, coords, neighbor_indices):
    coords5 = jnp.concatenate(
        [coords, jnp.zeros((N, 1), jnp.float32)], axis=1
    ).reshape(-1)
    nbr_t = neighbor_indices.T.reshape(-1)
    cov_t = _sc_cov(coords5, nbr_t).reshape(CC, N)

    rep = jnp.reshape(
        jnp.eye(F, dtype=jnp.float32)[:, :, None] * jnp.ones((1, 1, CC), jnp.float32),
        (F, F * CC),
    )
    sel = jnp.tile(jnp.eye(CC, dtype=jnp.float32), (1, F))
    return pl.pallas_call(
        _tc_weight_kernel,
        grid=(pl.cdiv(N, BN),),
        in_specs=[
            pl.BlockSpec((BN, F), lambda i: (i, 0)),
            pl.BlockSpec((CC, BN), lambda i: (0, i)),
            pl.BlockSpec((F, F * CC), lambda i: (0, 0)),
            pl.BlockSpec((CC, F * CC), lambda i: (0, 0)),
        ],
        out_specs=pl.BlockSpec((BN, F * CC), lambda i: (i, 0)),
        out_shape=jax.ShapeDtypeStruct((N, F * CC), jnp.float32),
    )(x, cov_t, rep, sel)


# speedup vs baseline: 13.2080x; 1.0076x over previous
import functools

import jax
import jax.numpy as jnp
from jax import lax
from jax.experimental import pallas as pl
from jax.experimental.pallas import tpu as pltpu
from jax.experimental.pallas import tpu_sc as plsc

N = 10000
F = 128
C = 4
CP = 5
K = 32
CC = C * C

NUM_WORKERS = 32
CHUNK = 336
STRIDE = 312
LANES = 16

BN = 1024


def _sc_cov_kernel(coords_hbm, nbr_hbm, out_hbm, coords_v, nbr_v, out_v, dsem):
    cid = lax.axis_index("c")
    sid = lax.axis_index("s")
    wid = sid * 2 + cid
    base = jnp.minimum(wid * STRIDE, N - CHUNK)

    copies = [pltpu.async_copy(coords_hbm, coords_v, dsem)]
    for kk in range(K):
        copies.append(pltpu.async_copy(
            nbr_hbm.at[pl.ds(kk * N + base, CHUNK)],
            nbr_v.at[pl.ds(kk * CHUNK, CHUNK)], dsem))
    for cp in copies:
        cp.wait()

    lane = lax.iota(jnp.int32, LANES)
    inv = jnp.float32(1.0 / (K - 1))

    def group_body(g, _):
        own5 = (base + g * LANES + lane) * CP
        m = [plsc.load_gather(coords_v, [own5 + c]) for c in range(C)]

        @plsc.parallel_loop(
            0, K, carry=tuple([jnp.zeros((LANES,), jnp.float32)] * 10), unroll=4
        )
        def acc(kk, acc_in):
            idx5 = nbr_v[pl.ds(kk * CHUNK + g * LANES, LANES)] * CP
            d = [plsc.load_gather(coords_v, [idx5 + c]) - m[c] for c in range(C)]
            new = []
            t = 0
            for a in range(C):
                for b in range(a, C):
                    new.append(acc_in[t] + d[a] * d[b])
                    t += 1
            return tuple(new)

        t = 0
        for a in range(C):
            for b in range(a, C):
                v = acc[t] * inv
                t += 1
                out_v[pl.ds((a * C + b) * CHUNK + g * LANES, LANES)] = v
                if a != b:
                    out_v[pl.ds((b * C + a) * CHUNK + g * LANES, LANES)] = v
        return 0

    lax.fori_loop(0, CHUNK // LANES, group_body, 0)
    stores = [pltpu.async_copy(
        out_v.at[pl.ds(j * CHUNK, CHUNK)],
        out_hbm.at[pl.ds(j * N + base, CHUNK)], dsem) for j in range(CC)]
    for st in stores:
        st.wait()


_sc_cov = functools.partial(
    pl.kernel,
    out_type=jax.ShapeDtypeStruct((CC * N,), jnp.float32),
    mesh=plsc.VectorSubcoreMesh(
        core_axis_name="c", subcore_axis_name="s", num_cores=2, num_subcores=16
    ),
    compiler_params=pltpu.CompilerParams(needs_layout_passes=False),
    scratch_types=[
        pltpu.VMEM((N * CP,), jnp.float32),
        pltpu.VMEM((K * CHUNK,), jnp.int32),
        pltpu.VMEM((CC * CHUNK,), jnp.float32),
        pltpu.SemaphoreType.DMA,
    ],
)(_sc_cov_kernel)


def _tc_weight_kernel(x_ref, cov_ref, rep_ref, sel_ref, out_ref):
    x = x_ref[...]
    covt = cov_ref[...]
    xr = jax.lax.dot(x, rep_ref[...], precision=jax.lax.Precision.DEFAULT)
    covq = jax.lax.dot_general(
        covt, sel_ref[...],
        dimension_numbers=(((0,), (0,)), ((), ())),
        precision=jax.lax.Precision.DEFAULT,
    )
    out_ref[...] = xr * covq


def kernel(x, coords, neighbor_indices):
    coords5 = jnp.concatenate(
        [coords, jnp.zeros((N, 1), jnp.float32)], axis=1
    ).reshape(-1)
    nbr_t = neighbor_indices.T.reshape(-1)
    cov_t = _sc_cov(coords5, nbr_t).reshape(CC, N)

    rep = jnp.reshape(
        jnp.eye(F, dtype=jnp.float32)[:, :, None] * jnp.ones((1, 1, CC), jnp.float32),
        (F, F * CC),
    )
    sel = jnp.tile(jnp.eye(CC, dtype=jnp.float32), (1, F))
    return pl.pallas_call(
        _tc_weight_kernel,
        grid=(pl.cdiv(N, BN),),
        in_specs=[
            pl.BlockSpec((BN, F), lambda i: (i, 0)),
            pl.BlockSpec((CC, BN), lambda i: (0, i)),
            pl.BlockSpec((F, F * CC), lambda i: (0, 0)),
            pl.BlockSpec((CC, F * CC), lambda i: (0, 0)),
        ],
        out_specs=pl.BlockSpec((BN, F * CC), lambda i: (i, 0)),
        out_shape=jax.ShapeDtypeStruct((N, F * CC), jnp.float32),
    )(x, cov_t, rep, sel)


# speedup vs baseline: 14.0399x; 1.0630x over previous
import functools

import jax
import jax.numpy as jnp
from jax import lax
from jax.experimental import pallas as pl
from jax.experimental.pallas import tpu as pltpu
from jax.experimental.pallas import tpu_sc as plsc

N = 10000
F = 128
C = 4
CP = 5
K = 32
CC = C * C

NUM_WORKERS = 32
CHUNK = 336
STRIDE = 312
LANES = 16

BN = 1024


def _sc_cov_kernel(coords_hbm, nbr_hbm, out_hbm, coords_sh, coords_v, nbr_v,
                   out_v, dsem):
    cid = lax.axis_index("c")
    sid = lax.axis_index("s")
    wid = sid * 2 + cid
    base = jnp.minimum(wid * STRIDE, N - CHUNK)

    copies = []
    for kk in range(K):
        copies.append(pltpu.async_copy(
            nbr_hbm.at[pl.ds(kk * N + base, CHUNK)],
            nbr_v.at[pl.ds(kk * CHUNK, CHUNK)], dsem))
    @pl.when(sid == 0)
    def _():
        pltpu.sync_copy(coords_hbm, coords_sh)
    plsc.subcore_barrier()
    copies.append(pltpu.async_copy(coords_sh, coords_v, dsem))
    for cp in copies:
        cp.wait()

    lane = lax.iota(jnp.int32, LANES)
    inv = jnp.float32(1.0 / (K - 1))

    def group_body(g, _):
        own5 = (base + g * LANES + lane) * CP
        m = [plsc.load_gather(coords_v, [own5 + c]) for c in range(C)]

        @plsc.parallel_loop(
            0, K, carry=tuple([jnp.zeros((LANES,), jnp.float32)] * 10), unroll=4
        )
        def acc(kk, acc_in):
            idx5 = nbr_v[pl.ds(kk * CHUNK + g * LANES, LANES)] * CP
            d = [plsc.load_gather(coords_v, [idx5 + c]) - m[c] for c in range(C)]
            new = []
            t = 0
            for a in range(C):
                for b in range(a, C):
                    new.append(acc_in[t] + d[a] * d[b])
                    t += 1
            return tuple(new)

        t = 0
        for a in range(C):
            for b in range(a, C):
                v = acc[t] * inv
                t += 1
                out_v[pl.ds((a * C + b) * CHUNK + g * LANES, LANES)] = v
                if a != b:
                    out_v[pl.ds((b * C + a) * CHUNK + g * LANES, LANES)] = v
        return 0

    lax.fori_loop(0, CHUNK // LANES, group_body, 0)
    stores = [pltpu.async_copy(
        out_v.at[pl.ds(j * CHUNK, CHUNK)],
        out_hbm.at[pl.ds(j * N + base, CHUNK)], dsem) for j in range(CC)]
    for st in stores:
        st.wait()


_sc_cov = functools.partial(
    pl.kernel,
    out_type=jax.ShapeDtypeStruct((CC * N,), jnp.float32),
    mesh=plsc.VectorSubcoreMesh(
        core_axis_name="c", subcore_axis_name="s", num_cores=2, num_subcores=16
    ),
    compiler_params=pltpu.CompilerParams(needs_layout_passes=False),
    scratch_types=[
        pltpu.VMEM_SHARED((N * CP,), jnp.float32),
        pltpu.VMEM((N * CP,), jnp.float32),
        pltpu.VMEM((K * CHUNK,), jnp.int32),
        pltpu.VMEM((CC * CHUNK,), jnp.float32),
        pltpu.SemaphoreType.DMA,
    ],
)(_sc_cov_kernel)


def _tc_weight_kernel(x_ref, cov_ref, rep_ref, sel_ref, out_ref):
    x = x_ref[...]
    covt = cov_ref[...]
    xr = jax.lax.dot(x, rep_ref[...], precision=jax.lax.Precision.DEFAULT)
    covq = jax.lax.dot_general(
        covt, sel_ref[...],
        dimension_numbers=(((0,), (0,)), ((), ())),
        precision=jax.lax.Precision.DEFAULT,
    )
    out_ref[...] = xr * covq


def kernel(x, coords, neighbor_indices):
    coords5 = jnp.concatenate(
        [coords, jnp.zeros((N, 1), jnp.float32)], axis=1
    ).reshape(-1)
    nbr_t = neighbor_indices.T.reshape(-1)
    cov_t = _sc_cov(coords5, nbr_t).reshape(CC, N)

    rep = jnp.reshape(
        jnp.eye(F, dtype=jnp.float32)[:, :, None] * jnp.ones((1, 1, CC), jnp.float32),
        (F, F * CC),
    )
    sel = jnp.tile(jnp.eye(CC, dtype=jnp.float32), (1, F))
    return pl.pallas_call(
        _tc_weight_kernel,
        grid=(pl.cdiv(N, BN),),
        in_specs=[
            pl.BlockSpec((BN, F), lambda i: (i, 0)),
            pl.BlockSpec((CC, BN), lambda i: (0, i)),
            pl.BlockSpec((F, F * CC), lambda i: (0, 0)),
            pl.BlockSpec((CC, F * CC), lambda i: (0, 0)),
        ],
        out_specs=pl.BlockSpec((BN, F * CC), lambda i: (i, 0)),
        out_shape=jax.ShapeDtypeStruct((N, F * CC), jnp.float32),
    )(x, cov_t, rep, sel)
